# Initial kernel scaffold; baseline (speedup 1.0000x reference)
import jax
import jax.numpy as jnp
from jax.experimental import pallas as pl


def kernel(features, edge_index, W, b):
    raise NotImplementedError("write your pallas kernel here")


# speedup vs baseline: 27.8844x; 27.8844x over previous
import functools

import jax
import jax.numpy as jnp
from jax import lax
from jax.experimental import pallas as pl
from jax.experimental.pallas import tpu as pltpu
from jax.experimental.pallas import tpu_sc as plsc

N = 10000
E = 320000
D = 128

NC = 2
NS = 16
NW = NC * NS
NPAD = 10240
EP = E // NW
CH = 128
NCH = -(-EP // CH)
EPP = NCH * CH
RPS = NPAD // NS

_MESH = plsc.VectorSubcoreMesh(
    core_axis_name="c", subcore_axis_name="s", num_cores=NC, num_subcores=NS
)


def _deg_body(sidx, didx, degp, sv, dv, ones_v, zv, d0, d1):
    c = lax.axis_index("c")
    s = lax.axis_index("s")
    wid = s * NC + c

    def _ones(i, _):
        ones_v[pl.ds(i * 16, 16)] = jnp.ones((16,), jnp.float32)
        return 0

    lax.fori_loop(0, CH // 16, _ones, 0)

    def _zeros(i, _):
        zv[pl.ds(i * 16, 16)] = jnp.zeros((16,), jnp.float32)
        return 0

    lax.fori_loop(0, RPS // 16, _zeros, 0)
    pltpu.sync_copy(zv, d0.at[pl.ds(s * RPS, RPS)])
    pltpu.sync_copy(zv, d1.at[pl.ds(s * RPS, RPS)])
    pltpu.sync_copy(sidx.at[wid], sv)
    pltpu.sync_copy(didx.at[wid], dv)
    plsc.subcore_barrier()

    def _scat(j, _):
        pltpu.sync_copy(ones_v, d0.at[sv.at[j]], add=True)
        pltpu.sync_copy(ones_v, d1.at[dv.at[j]], add=True)
        return 0

    lax.fori_loop(0, NCH, _scat, 0)
    plsc.subcore_barrier()
    pltpu.sync_copy(d0.at[pl.ds(s * RPS, RPS)], degp.at[c, 0, pl.ds(s * RPS, RPS)])
    pltpu.sync_copy(d1.at[pl.ds(s * RPS, RPS)], degp.at[c, 1, pl.ds(s * RPS, RPS)])


_deg_call = functools.partial(
    pl.kernel,
    out_type=jax.ShapeDtypeStruct((NC, 2, NPAD), jnp.float32),
    mesh=_MESH,
    scratch_types=[
        pltpu.VMEM((NCH, CH), jnp.int32),
        pltpu.VMEM((NCH, CH), jnp.int32),
        pltpu.VMEM((CH,), jnp.float32),
        pltpu.VMEM((RPS,), jnp.float32),
        pltpu.VMEM_SHARED((NPAD,), jnp.float32),
        pltpu.VMEM_SHARED((NPAD,), jnp.float32),
    ],
)(_deg_body)


def _scatter_body(h, sidx, didx, part, sv, dv, rows, zb, acc):
    c = lax.axis_index("c")
    s = lax.axis_index("s")
    wid = s * NC + c

    def _zb(i, _):
        for k in range(D // 16):
            zb[i, pl.ds(k * 16, 16)] = jnp.zeros((16,), jnp.float32)
        return 0

    lax.fori_loop(0, 64, _zb, 0)
    for k in range(RPS // 64):
        pltpu.sync_copy(zb, acc.at[pl.ds(s * RPS + k * 64, 64)])
    pltpu.sync_copy(sidx.at[wid], sv)
    pltpu.sync_copy(didx.at[wid], dv)
    plsc.subcore_barrier()

    def _edge(j, _):
        pltpu.sync_copy(h.at[sv.at[j]], rows)
        pltpu.sync_copy(rows, acc.at[dv.at[j]], add=True)
        return 0

    lax.fori_loop(0, NCH, _edge, 0)
    plsc.subcore_barrier()
    pltpu.sync_copy(acc.at[pl.ds(s * RPS, RPS)], part.at[c, pl.ds(s * RPS, RPS)])


_scatter_call = functools.partial(
    pl.kernel,
    out_type=jax.ShapeDtypeStruct((NC, NPAD, D), jnp.float32),
    mesh=_MESH,
    scratch_types=[
        pltpu.VMEM((NCH, CH), jnp.int32),
        pltpu.VMEM((NCH, CH), jnp.int32),
        pltpu.VMEM((CH, D), jnp.float32),
        pltpu.VMEM((64, D), jnp.float32),
        pltpu.VMEM_SHARED((NPAD, D), jnp.float32),
    ],
)(_scatter_body)


def _mm_body(x_ref, w_ref, b_ref, deg_ref, o_ref):
    h = jnp.dot(x_ref[...], w_ref[...], preferred_element_type=jnp.float32)
    h = h + b_ref[...]
    dsum = deg_ref[0, :] + deg_ref[2, :]
    o_ref[...] = h * lax.rsqrt(jnp.clip(dsum, 1.0, None))[:, None]


_BN1 = 1024
_mm_call = pl.pallas_call(
    _mm_body,
    grid=(NPAD // _BN1,),
    in_specs=[
        pl.BlockSpec((_BN1, D), lambda j: (j, 0)),
        pl.BlockSpec((D, D), lambda j: (0, 0)),
        pl.BlockSpec((1, D), lambda j: (0, 0)),
        pl.BlockSpec((2 * NC, _BN1), lambda j: (0, j)),
    ],
    out_specs=pl.BlockSpec((_BN1, D), lambda j: (j, 0)),
    out_shape=jax.ShapeDtypeStruct((NPAD, D), jnp.float32),
)


def _comb_body(p_ref, deg_ref, o_ref):
    ssum = p_ref[0] + p_ref[1]
    dsum = deg_ref[1, :] + deg_ref[3, :]
    o_ref[...] = ssum * lax.rsqrt(jnp.clip(dsum, 1.0, None))[:, None]


_BN2 = 1024
_comb_call = pl.pallas_call(
    _comb_body,
    grid=(NPAD // _BN2,),
    in_specs=[
        pl.BlockSpec((NC, _BN2, D), lambda j: (0, j, 0)),
        pl.BlockSpec((2 * NC, _BN2), lambda j: (0, j)),
    ],
    out_specs=pl.BlockSpec((_BN2, D), lambda j: (j, 0)),
    out_shape=jax.ShapeDtypeStruct((NPAD, D), jnp.float32),
)


def kernel(features, edge_index, W, b):
    src = edge_index[0].reshape(NW, EP)
    dst = edge_index[1].reshape(NW, EP)
    pad = N + (jnp.arange(EPP - EP, dtype=jnp.int32) % (NPAD - N))
    pad = jnp.broadcast_to(pad[None, :], (NW, EPP - EP))
    sidx = jnp.concatenate([src, pad], axis=1).reshape(NW, NCH, CH)
    didx = jnp.concatenate([dst, pad], axis=1).reshape(NW, NCH, CH)
    xp = jnp.concatenate(
        [features, jnp.zeros((NPAD - N, D), jnp.float32)], axis=0
    )

    degp = _deg_call(sidx, didx).reshape(2 * NC, NPAD)
    hs = _mm_call(xp, W, b.reshape(1, D), degp)
    part = _scatter_call(hs, sidx, didx)
    return _comb_call(part, degp)[:N]
